# Initial kernel scaffold; baseline (speedup 1.0000x reference)
import jax
import jax.numpy as jnp
from jax.experimental import pallas as pl


def kernel(x):
    raise NotImplementedError("write your pallas kernel here")


# speedup vs baseline: 45.0917x; 45.0917x over previous
import jax
import jax.numpy as jnp
from jax.experimental import pallas as pl
from jax.experimental.pallas import tpu as pltpu

_GS = 8
_NZ = 4
_DELTA = 127.0


def _maxabs_body(x_ref, o_ref):
    i = pl.program_id(0)
    m = jnp.max(jnp.abs(x_ref[...]))

    @pl.when(i == 0)
    def _init():
        o_ref[0, 0] = m

    @pl.when(i > 0)
    def _acc():
        o_ref[0, 0] = jnp.maximum(o_ref[0, 0], m)


def _quant_body(tmax_ref, x_ref, o_ref):
    t = jnp.tanh(x_ref[...])
    y = t / tmax_ref[0, 0]
    q = jnp.round(y * _DELTA)
    a = jnp.abs(q)
    cidx = jax.lax.broadcasted_iota(jnp.int32, a.shape, 1)
    rank = jnp.zeros(a.shape, jnp.int32)
    for j in range(_GS):
        aj = a[:, j : j + 1, :]
        cond = (aj < a) | ((aj == a) & (cidx > j))
        rank = rank + cond.astype(jnp.int32)
    o_ref[...] = jnp.where(rank >= _NZ, q, 0.0) / _DELTA


def kernel(x):
    out_c, in_c, h, w = x.shape
    hw = h * w
    g = in_c // _GS
    n = out_c * g

    total = out_c * in_c * hw
    lanes = 1024
    rows = total // lanes
    b1 = 288
    assert rows % b1 == 0
    flat = x.reshape(rows, lanes)
    maxabs = pl.pallas_call(
        _maxabs_body,
        grid=(rows // b1,),
        in_specs=[pl.BlockSpec((b1, lanes), lambda i: (i, 0))],
        out_specs=pl.BlockSpec(memory_space=pltpu.SMEM),
        out_shape=jax.ShapeDtypeStruct((1, 1), jnp.float32),
    )(flat)
    tmax = jnp.tanh(maxabs)

    b2 = 64
    assert n % b2 == 0 and in_c % _GS == 0
    xr = x.reshape(n, _GS, hw)
    out = pl.pallas_call(
        _quant_body,
        grid=(n // b2,),
        in_specs=[
            pl.BlockSpec(memory_space=pltpu.SMEM),
            pl.BlockSpec((b2, _GS, hw), lambda i: (i, 0, 0)),
        ],
        out_specs=pl.BlockSpec((b2, _GS, hw), lambda i: (i, 0, 0)),
        out_shape=jax.ShapeDtypeStruct((n, _GS, hw), jnp.float32),
    )(tmax, xr)
    return out.reshape(x.shape)


# speedup vs baseline: 205.6407x; 4.5605x over previous
import jax
import jax.numpy as jnp
from jax.experimental import pallas as pl
from jax.experimental.pallas import tpu as pltpu

_GS = 8
_NZ = 4
_DELTA = 127.0


def _maxabs_body(x_ref, o_ref):
    i = pl.program_id(0)
    m = jnp.max(jnp.abs(x_ref[...]))

    @pl.when(i == 0)
    def _init():
        o_ref[0, 0] = m

    @pl.when(i > 0)
    def _acc():
        o_ref[0, 0] = jnp.maximum(o_ref[0, 0], m)


def _quant_body(tmax_ref, x_ref, o_ref):
    t = jnp.tanh(x_ref[...])
    y = t / tmax_ref[0, 0]
    q = jnp.round(y * _DELTA)
    a = jnp.abs(q)
    shape = a.shape
    l8 = jax.lax.broadcasted_iota(jnp.int32, shape, 2) % _GS
    rank = jnp.zeros(shape, jnp.int32)
    for d in range(1, _GS):
        wrap = (l8 + d) >= _GS
        b = jnp.where(wrap, jnp.roll(a, _GS - d, axis=2), jnp.roll(a, -d, axis=2))
        cond = (b < a) | ((b == a) & wrap)
        rank = rank + cond.astype(jnp.int32)
    o_ref[...] = jnp.where(rank >= _NZ, q, 0.0) / _DELTA


def kernel(x):
    out_c, in_c, h, w = x.shape
    hw = h * w
    xt = jnp.transpose(x, (2, 3, 0, 1)).reshape(hw, out_c, in_c)

    b1 = 2
    maxabs = pl.pallas_call(
        _maxabs_body,
        grid=(hw // b1,),
        in_specs=[pl.BlockSpec((b1, out_c, in_c), lambda i: (i, 0, 0))],
        out_specs=pl.BlockSpec(memory_space=pltpu.SMEM),
        out_shape=jax.ShapeDtypeStruct((1, 1), jnp.float32),
    )(xt)
    tmax = jnp.tanh(maxabs)

    b2h, b2c = 2, 128
    out = pl.pallas_call(
        _quant_body,
        grid=(hw // b2h, out_c // b2c),
        in_specs=[
            pl.BlockSpec(memory_space=pltpu.SMEM),
            pl.BlockSpec((b2h, b2c, in_c), lambda i, j: (i, j, 0)),
        ],
        out_specs=pl.BlockSpec((b2h, b2c, in_c), lambda i, j: (i, j, 0)),
        out_shape=jax.ShapeDtypeStruct((hw, out_c, in_c), jnp.float32),
    )(tmax, xt)
    return jnp.transpose(out.reshape(h, w, out_c, in_c), (2, 3, 0, 1))


# speedup vs baseline: 270.9033x; 1.3174x over previous
import jax
import jax.numpy as jnp
from jax.experimental import pallas as pl
from jax.experimental.pallas import tpu as pltpu

_GS = 8
_NZ = 4
_DELTA = 127.0


def _maxabs_body(x_ref, o_ref):
    i = pl.program_id(0)
    m = jnp.max(jnp.abs(x_ref[...]))

    @pl.when(i == 0)
    def _init():
        o_ref[0, 0] = m

    @pl.when(i > 0)
    def _acc():
        o_ref[0, 0] = jnp.maximum(o_ref[0, 0], m)


def _quant_body(tmax_ref, x_ref, o_ref):
    t = jnp.tanh(x_ref[...])
    y = t / tmax_ref[0, 0]
    q = jnp.round(y * _DELTA)
    shape = q.shape
    l8 = jax.lax.broadcasted_iota(jnp.int32, shape, 2) % _GS
    key = jnp.abs(q) * float(_GS) + l8.astype(jnp.float32)
    rank = jnp.zeros(shape, jnp.float32)
    for d in range(1, _GS):
        wrap = (l8 + d) >= _GS
        b = jnp.where(wrap, jnp.roll(key, _GS - d, axis=2),
                      jnp.roll(key, -d, axis=2))
        rank = rank + jnp.where(b < key, 1.0, 0.0)
    o_ref[...] = jnp.where(rank >= float(_NZ), q, 0.0) / _DELTA


def kernel(x):
    out_c, in_c, h, w = x.shape
    hw = h * w
    xt = jnp.transpose(x, (2, 3, 0, 1)).reshape(hw, out_c, in_c)

    b1 = 2
    maxabs = pl.pallas_call(
        _maxabs_body,
        grid=(hw // b1,),
        in_specs=[pl.BlockSpec((b1, out_c, in_c), lambda i: (i, 0, 0))],
        out_specs=pl.BlockSpec(memory_space=pltpu.SMEM),
        out_shape=jax.ShapeDtypeStruct((1, 1), jnp.float32),
    )(xt)
    tmax = jnp.tanh(maxabs)

    b2h, b2c = 2, 128
    out = pl.pallas_call(
        _quant_body,
        grid=(hw // b2h, out_c // b2c),
        in_specs=[
            pl.BlockSpec(memory_space=pltpu.SMEM),
            pl.BlockSpec((b2h, b2c, in_c), lambda i, j: (i, j, 0)),
        ],
        out_specs=pl.BlockSpec((b2h, b2c, in_c), lambda i, j: (i, j, 0)),
        out_shape=jax.ShapeDtypeStruct((hw, out_c, in_c), jnp.float32),
    )(tmax, xt)
    return jnp.transpose(out.reshape(h, w, out_c, in_c), (2, 3, 0, 1))
